# baseline (speedup 1.0000x reference)
import functools

import jax
import jax.numpy as jnp
from jax import lax
from jax.experimental import pallas as pl
from jax.experimental.pallas import tpu as pltpu
from jax.experimental.pallas import tpu_sc as plsc

N = 10000
E = 320000
IN = 128
H = 128
L = 6
G = 64

NW = 32
K = 160
NCH = 64
NPH = 4
NCHP = NCH // NPH
NBUF = 2
EPW = NCH * K
E_PAD = NW * EPW
N_P = N + 8
RPT = 624

R = 1000
NBLK = N // R


def _dotT(a, w):
    return lax.dot_general(a, w, (((1,), (1,)), ((), ())),
                           preferred_element_type=jnp.float32)


def _zero_tail(P_blk):
    return jnp.where(pl.program_id(0) == NBLK, 0.0, P_blk)


def _enc_msg_body(x_ref, encW_ref, encb_ref, W1_ref, b1_ref, W2_ref, b2_ref,
                  h_ref, P_ref):
    h = _dotT(x_ref[...], encW_ref[...]) + encb_ref[...]
    h_ref[...] = h
    hid = jnp.maximum(_dotT(h, W1_ref[...]) + b1_ref[...], 0.0)
    P_ref[...] = _zero_tail(_dotT(hid, W2_ref[...]) + b2_ref[...])


def _gru_core(m0_ref, m1_ref, P_ref, h_ref, Wih_ref, bih_ref, Whh_ref,
              bhh_ref):
    m = m0_ref[...] + m1_ref[...] - P_ref[...]
    h = h_ref[...]
    gi = _dotT(m, Wih_ref[...]) + bih_ref[...]
    gh = _dotT(h, Whh_ref[...]) + bhh_ref[...]
    r = jax.nn.sigmoid(gi[:, :H] + gh[:, :H])
    z = jax.nn.sigmoid(gi[:, H:2 * H] + gh[:, H:2 * H])
    n = jnp.tanh(gi[:, 2 * H:] + r * gh[:, 2 * H:])
    return (1.0 - z) * n + z * h


def _gru_msg_body(m0_ref, m1_ref, P_ref, h_ref, Wih_ref, bih_ref, Whh_ref,
                  bhh_ref, W1_ref, b1_ref, W2_ref, b2_ref, hout_ref, Pout_ref):
    hn = _gru_core(m0_ref, m1_ref, P_ref, h_ref, Wih_ref, bih_ref, Whh_ref,
                   bhh_ref)
    hout_ref[...] = hn
    hid = jnp.maximum(_dotT(hn, W1_ref[...]) + b1_ref[...], 0.0)
    Pout_ref[...] = _zero_tail(_dotT(hid, W2_ref[...]) + b2_ref[...])


def _gru_pool_body(m0_ref, m1_ref, P_ref, h_ref, Wih_ref, bih_ref, Whh_ref,
                   bhh_ref, batch_ref, W1_ref, b1_ref, W2_ref, b2_ref,
                   out_ref, sums_ref, cnts_ref):
    i = pl.program_id(0)
    hn = _gru_core(m0_ref, m1_ref, P_ref, h_ref, Wih_ref, bih_ref, Whh_ref,
                   bhh_ref)
    b = batch_ref[...]
    gids = lax.broadcasted_iota(jnp.int32, (1, G), 1)
    oh = (b == gids).astype(jnp.float32)
    s = lax.dot_general(oh, hn, (((0,), (0,)), ((), ())),
                        preferred_element_type=jnp.float32)
    c = lax.dot_general(oh, jnp.ones((R, 1), jnp.float32),
                        (((0,), (0,)), ((), ())),
                        preferred_element_type=jnp.float32)

    @pl.when(i == 0)
    def _():
        sums_ref[...] = s
        cnts_ref[...] = c

    @pl.when(i > 0)
    def _():
        sums_ref[...] += s
        cnts_ref[...] += c

    @pl.when(i == NBLK - 1)
    def _():
        pooled = sums_ref[...] / jnp.maximum(cnts_ref[...], 1.0)
        ph = jnp.maximum(_dotT(pooled, W1_ref[...]) + b1_ref[...], 0.0)
        out_ref[...] = (jnp.sum(ph * W2_ref[...], axis=1, keepdims=True)
                        + b2_ref[0, 0])


def _row_spec(shape):
    return pl.BlockSpec(shape, lambda i: (i, 0))


def _crow_spec(shape):
    return pl.BlockSpec(shape, lambda i: (jnp.minimum(i, NBLK - 1), 0))


def _prow_spec(shape):
    return pl.BlockSpec(shape, lambda i: (i, 0))


def _full_spec(shape):
    return pl.BlockSpec(shape, lambda i: (0, 0))


def _enc_msg(x, encW, encb, W1, b1, W2, b2):
    return pl.pallas_call(
        _enc_msg_body,
        grid=(NBLK + 1,),
        in_specs=[
            _crow_spec((R, IN)),
            _full_spec((H, IN)), _full_spec((1, H)),
            _full_spec((H, H)), _full_spec((1, H)),
            _full_spec((H, H)), _full_spec((1, H)),
        ],
        out_specs=[_crow_spec((R, H)), _prow_spec((R, H))],
        out_shape=[jax.ShapeDtypeStruct((N, H), jnp.float32),
                   jax.ShapeDtypeStruct((N_P, H), jnp.float32)],
    )(x, encW, encb, W1, b1, W2, b2)


def _gru_msg(m0, m1, P, h, Wih, bih, Whh, bhh, W1, b1, W2, b2):
    return pl.pallas_call(
        _gru_msg_body,
        grid=(NBLK + 1,),
        in_specs=[
            _crow_spec((R, H)), _crow_spec((R, H)), _crow_spec((R, H)),
            _crow_spec((R, H)),
            _full_spec((3 * H, H)), _full_spec((1, 3 * H)),
            _full_spec((3 * H, H)), _full_spec((1, 3 * H)),
            _full_spec((H, H)), _full_spec((1, H)),
            _full_spec((H, H)), _full_spec((1, H)),
        ],
        out_specs=[_crow_spec((R, H)), _prow_spec((R, H))],
        out_shape=[jax.ShapeDtypeStruct((N, H), jnp.float32),
                   jax.ShapeDtypeStruct((N_P, H), jnp.float32)],
    )(m0, m1, P, h, Wih, bih, Whh, bhh, W1, b1, W2, b2)


def _gru_pool(m0, m1, P, h, Wih, bih, Whh, bhh, batch2, W1, b1, W2, b2):
    return pl.pallas_call(
        _gru_pool_body,
        grid=(NBLK,),
        in_specs=[
            _row_spec((R, H)), _row_spec((R, H)), _row_spec((R, H)),
            _row_spec((R, H)),
            _full_spec((3 * H, H)), _full_spec((1, 3 * H)),
            _full_spec((3 * H, H)), _full_spec((1, 3 * H)),
            _row_spec((R, 1)),
            _full_spec((H // 2, H)), _full_spec((1, H // 2)),
            _full_spec((1, H // 2)), _full_spec((1, 1)),
        ],
        out_specs=_full_spec((G, 1)),
        out_shape=jax.ShapeDtypeStruct((G, 1), jnp.float32),
        scratch_shapes=[pltpu.VMEM((G, H), jnp.float32),
                        pltpu.VMEM((G, 1), jnp.float32)],
    )(m0, m1, P, h, Wih, bih, Whh, bhh, batch2, W1, b1, W2, b2)


def _make_scatter():
    mesh = plsc.VectorSubcoreMesh(core_axis_name="c", subcore_axis_name="s")

    @functools.partial(
        pl.kernel,
        out_type=jax.ShapeDtypeStruct((2, N, H), jnp.float32),
        mesh=mesh,
        scratch_types=[
            pltpu.VMEM_SHARED((N, H), jnp.float32),
            pltpu.VMEM((NCHP * K,), jnp.int32),
            pltpu.VMEM((NCHP * K,), jnp.int32),
            pltpu.VMEM((NBUF, K, H), jnp.float32),
            [pltpu.SemaphoreType.DMA] * NBUF,
            pltpu.SemaphoreType.DMA,
        ],
    )
    def scatter_kernel(P_hbm, src_hbm, dst_hbm, m_hbm, acc, src_v, dst_v,
                       rows_v, gsems, isem):
        cid = lax.axis_index("c")
        sid = lax.axis_index("s")
        wid = sid * 2 + cid
        r0 = pl.multiple_of(sid * RPT, 8)
        pltpu.async_copy(P_hbm.at[pl.ds(r0, RPT)], acc.at[pl.ds(r0, RPT)],
                         isem)

        @pl.when(sid == 15)
        def _():
            pltpu.async_copy(P_hbm.at[pl.ds(16 * RPT, N - 16 * RPT)],
                             acc.at[pl.ds(16 * RPT, N - 16 * RPT)], isem)

        def fire_gather(j, b):
            jb = pl.multiple_of(j * K, 8)
            pltpu.async_copy(P_hbm.at[src_v.at[pl.ds(jb, K)]], rows_v.at[b],
                             gsems[b])

        def wait_gather(j, b):
            jb = pl.multiple_of(j * K, 8)
            pltpu.make_async_copy(P_hbm.at[src_v.at[pl.ds(jb, K)]],
                                  rows_v.at[b], gsems[b]).wait()

        def scatter(j, b):
            jb = pl.multiple_of(j * K, 8)
            pltpu.sync_copy(rows_v.at[b], acc.at[dst_v.at[pl.ds(jb, K)]],
                            add=True)

        e0 = wid * EPW
        for p in range(NPH):
            off = pl.multiple_of(e0 + p * NCHP * K, 8)
            pltpu.sync_copy(src_hbm.at[pl.ds(off, NCHP * K)], src_v)
            pltpu.sync_copy(dst_hbm.at[pl.ds(off, NCHP * K)], dst_v)
            for b in range(NBUF):
                fire_gather(b, b)
            if p == 0:
                pltpu.make_async_copy(P_hbm.at[pl.ds(r0, RPT)],
                                      acc.at[pl.ds(r0, RPT)], isem).wait()

                @pl.when(sid == 15)
                def _():
                    pltpu.make_async_copy(
                        P_hbm.at[pl.ds(16 * RPT, N - 16 * RPT)],
                        acc.at[pl.ds(16 * RPT, N - 16 * RPT)], isem).wait()

                plsc.subcore_barrier()

            def body(jo, carry):
                for b in range(NBUF):
                    j = jo * NBUF + b
                    wait_gather(j, b)
                    scatter(j, b)

                    @pl.when(j + NBUF < NCHP)
                    def _():
                        fire_gather(j + NBUF, b)
                return carry

            lax.fori_loop(0, NCHP // NBUF, body, 0)
        plsc.subcore_barrier()
        pltpu.sync_copy(acc.at[pl.ds(r0, RPT)], m_hbm.at[cid, pl.ds(r0, RPT)])

        @pl.when(sid == 15)
        def _():
            pltpu.sync_copy(acc.at[pl.ds(16 * RPT, N - 16 * RPT)],
                            m_hbm.at[cid, pl.ds(16 * RPT, N - 16 * RPT)])

    return scatter_kernel


@functools.lru_cache(maxsize=None)
def _get_scatter():
    return _make_scatter()


def _scatter_edges(P, src_p, dst_p):
    return _get_scatter()(P, src_p, dst_p)


def kernel(x, edge_index, batch, enc_W, enc_b, msg_W1, msg_b1, msg_W2, msg_b2,
           gru_Wih, gru_bih, gru_Whh, gru_bhh, head_W1, head_b1, head_W2,
           head_b2):
    pad = E_PAD - E
    pidx = jnp.arange(pad, dtype=jnp.int32)
    src_p = jnp.concatenate([edge_index[0], N + (pidx % (N_P - N))])
    dst_p = jnp.concatenate([edge_index[1], pidx % N])

    h, P = _enc_msg(x, enc_W, enc_b.reshape(1, H),
                    msg_W1[0], msg_b1[0].reshape(1, H),
                    msg_W2[0], msg_b2[0].reshape(1, H))
    for l in range(L):
        m = _scatter_edges(P, src_p, dst_p)
        gru_args = (m[0], m[1], P, h,
                    gru_Wih[l], gru_bih[l].reshape(1, 3 * H),
                    gru_Whh[l], gru_bhh[l].reshape(1, 3 * H))
        if l < L - 1:
            h, P = _gru_msg(*gru_args,
                            msg_W1[l + 1], msg_b1[l + 1].reshape(1, H),
                            msg_W2[l + 1], msg_b2[l + 1].reshape(1, H))
        else:
            out = _gru_pool(*gru_args, batch.reshape(N, 1),
                            head_W1, head_b1.reshape(1, H // 2),
                            head_W2, head_b2.reshape(1, 1))
    return out.reshape(G)


# speedup vs baseline: 1.0335x; 1.0335x over previous
import functools

import jax
import jax.numpy as jnp
from jax import lax
from jax.experimental import pallas as pl
from jax.experimental.pallas import tpu as pltpu
from jax.experimental.pallas import tpu_sc as plsc

N = 10000
E = 320000
IN = 128
H = 128
L = 6
G = 64

NW = 32
K = 128
NCH = 80
NPH = 2
NCHP = NCH // NPH
NBUF = 2
EPW = NCH * K
E_PAD = NW * EPW
N_P = N + 8
RPT = 624

R = 1000
NBLK = N // R


def _dotT(a, w):
    return lax.dot_general(a, w, (((1,), (1,)), ((), ())),
                           preferred_element_type=jnp.float32)


def _zero_tail(P_blk):
    return jnp.where(pl.program_id(0) == NBLK, 0.0, P_blk)


def _enc_msg_body(x_ref, encW_ref, encb_ref, W1_ref, b1_ref, W2_ref, b2_ref,
                  h_ref, P_ref):
    h = _dotT(x_ref[...], encW_ref[...]) + encb_ref[...]
    h_ref[...] = h
    hid = jnp.maximum(_dotT(h, W1_ref[...]) + b1_ref[...], 0.0)
    P_ref[...] = _zero_tail(_dotT(hid, W2_ref[...]) + b2_ref[...])


def _gru_core(m0_ref, m1_ref, P_ref, h_ref, Wih_ref, bih_ref, Whh_ref,
              bhh_ref):
    m = m0_ref[...] + m1_ref[...] - P_ref[...]
    h = h_ref[...]
    gi = _dotT(m, Wih_ref[...]) + bih_ref[...]
    gh = _dotT(h, Whh_ref[...]) + bhh_ref[...]
    r = jax.nn.sigmoid(gi[:, :H] + gh[:, :H])
    z = jax.nn.sigmoid(gi[:, H:2 * H] + gh[:, H:2 * H])
    n = jnp.tanh(gi[:, 2 * H:] + r * gh[:, 2 * H:])
    return (1.0 - z) * n + z * h


def _gru_msg_body(m0_ref, m1_ref, P_ref, h_ref, Wih_ref, bih_ref, Whh_ref,
                  bhh_ref, W1_ref, b1_ref, W2_ref, b2_ref, hout_ref, Pout_ref):
    hn = _gru_core(m0_ref, m1_ref, P_ref, h_ref, Wih_ref, bih_ref, Whh_ref,
                   bhh_ref)
    hout_ref[...] = hn
    hid = jnp.maximum(_dotT(hn, W1_ref[...]) + b1_ref[...], 0.0)
    Pout_ref[...] = _zero_tail(_dotT(hid, W2_ref[...]) + b2_ref[...])


def _gru_pool_body(m0_ref, m1_ref, P_ref, h_ref, Wih_ref, bih_ref, Whh_ref,
                   bhh_ref, batch_ref, W1_ref, b1_ref, W2_ref, b2_ref,
                   out_ref, sums_ref, cnts_ref):
    i = pl.program_id(0)
    hn = _gru_core(m0_ref, m1_ref, P_ref, h_ref, Wih_ref, bih_ref, Whh_ref,
                   bhh_ref)
    b = batch_ref[...]
    gids = lax.broadcasted_iota(jnp.int32, (1, G), 1)
    oh = (b == gids).astype(jnp.float32)
    s = lax.dot_general(oh, hn, (((0,), (0,)), ((), ())),
                        preferred_element_type=jnp.float32)
    c = lax.dot_general(oh, jnp.ones((R, 1), jnp.float32),
                        (((0,), (0,)), ((), ())),
                        preferred_element_type=jnp.float32)

    @pl.when(i == 0)
    def _():
        sums_ref[...] = s
        cnts_ref[...] = c

    @pl.when(i > 0)
    def _():
        sums_ref[...] += s
        cnts_ref[...] += c

    @pl.when(i == NBLK - 1)
    def _():
        pooled = sums_ref[...] / jnp.maximum(cnts_ref[...], 1.0)
        ph = jnp.maximum(_dotT(pooled, W1_ref[...]) + b1_ref[...], 0.0)
        out_ref[...] = (jnp.sum(ph * W2_ref[...], axis=1, keepdims=True)
                        + b2_ref[0, 0])


def _row_spec(shape):
    return pl.BlockSpec(shape, lambda i: (i, 0))


def _crow_spec(shape):
    return pl.BlockSpec(shape, lambda i: (jnp.minimum(i, NBLK - 1), 0))


def _prow_spec(shape):
    return pl.BlockSpec(shape, lambda i: (i, 0))


def _full_spec(shape):
    return pl.BlockSpec(shape, lambda i: (0, 0))


def _enc_msg(x, encW, encb, W1, b1, W2, b2):
    return pl.pallas_call(
        _enc_msg_body,
        grid=(NBLK + 1,),
        in_specs=[
            _crow_spec((R, IN)),
            _full_spec((H, IN)), _full_spec((1, H)),
            _full_spec((H, H)), _full_spec((1, H)),
            _full_spec((H, H)), _full_spec((1, H)),
        ],
        out_specs=[_crow_spec((R, H)), _prow_spec((R, H))],
        out_shape=[jax.ShapeDtypeStruct((N, H), jnp.float32),
                   jax.ShapeDtypeStruct((N_P, H), jnp.float32)],
    )(x, encW, encb, W1, b1, W2, b2)


def _gru_msg(m0, m1, P, h, Wih, bih, Whh, bhh, W1, b1, W2, b2):
    return pl.pallas_call(
        _gru_msg_body,
        grid=(NBLK + 1,),
        in_specs=[
            _crow_spec((R, H)), _crow_spec((R, H)), _crow_spec((R, H)),
            _crow_spec((R, H)),
            _full_spec((3 * H, H)), _full_spec((1, 3 * H)),
            _full_spec((3 * H, H)), _full_spec((1, 3 * H)),
            _full_spec((H, H)), _full_spec((1, H)),
            _full_spec((H, H)), _full_spec((1, H)),
        ],
        out_specs=[_crow_spec((R, H)), _prow_spec((R, H))],
        out_shape=[jax.ShapeDtypeStruct((N, H), jnp.float32),
                   jax.ShapeDtypeStruct((N_P, H), jnp.float32)],
    )(m0, m1, P, h, Wih, bih, Whh, bhh, W1, b1, W2, b2)


def _gru_pool(m0, m1, P, h, Wih, bih, Whh, bhh, batch2, W1, b1, W2, b2):
    return pl.pallas_call(
        _gru_pool_body,
        grid=(NBLK,),
        in_specs=[
            _row_spec((R, H)), _row_spec((R, H)), _row_spec((R, H)),
            _row_spec((R, H)),
            _full_spec((3 * H, H)), _full_spec((1, 3 * H)),
            _full_spec((3 * H, H)), _full_spec((1, 3 * H)),
            _row_spec((R, 1)),
            _full_spec((H // 2, H)), _full_spec((1, H // 2)),
            _full_spec((1, H // 2)), _full_spec((1, 1)),
        ],
        out_specs=_full_spec((G, 1)),
        out_shape=jax.ShapeDtypeStruct((G, 1), jnp.float32),
        scratch_shapes=[pltpu.VMEM((G, H), jnp.float32),
                        pltpu.VMEM((G, 1), jnp.float32)],
    )(m0, m1, P, h, Wih, bih, Whh, bhh, batch2, W1, b1, W2, b2)


def _make_scatter():
    mesh = plsc.VectorSubcoreMesh(core_axis_name="c", subcore_axis_name="s")

    @functools.partial(
        pl.kernel,
        out_type=jax.ShapeDtypeStruct((2, N, H), jnp.float32),
        mesh=mesh,
        scratch_types=[
            pltpu.VMEM_SHARED((N, H), jnp.float32),
            pltpu.VMEM((NCHP * K,), jnp.int32),
            pltpu.VMEM((NCHP * K,), jnp.int32),
            pltpu.VMEM((NBUF, K, H), jnp.float32),
            [pltpu.SemaphoreType.DMA] * NBUF,
            pltpu.SemaphoreType.DMA,
        ],
    )
    def scatter_kernel(P_hbm, src_hbm, dst_hbm, m_hbm, acc, src_v, dst_v,
                       rows_v, gsems, isem):
        cid = lax.axis_index("c")
        sid = lax.axis_index("s")
        wid = sid * 2 + cid
        r0 = pl.multiple_of(sid * RPT, 8)
        pltpu.async_copy(P_hbm.at[pl.ds(r0, RPT)], acc.at[pl.ds(r0, RPT)],
                         isem)

        @pl.when(sid == 15)
        def _():
            pltpu.async_copy(P_hbm.at[pl.ds(16 * RPT, N - 16 * RPT)],
                             acc.at[pl.ds(16 * RPT, N - 16 * RPT)], isem)

        def fire_gather(j, b):
            jb = pl.multiple_of(j * K, 8)
            pltpu.async_copy(P_hbm.at[src_v.at[pl.ds(jb, K)]], rows_v.at[b],
                             gsems[b])

        def wait_gather(j, b):
            jb = pl.multiple_of(j * K, 8)
            pltpu.make_async_copy(P_hbm.at[src_v.at[pl.ds(jb, K)]],
                                  rows_v.at[b], gsems[b]).wait()

        def scatter(j, b):
            jb = pl.multiple_of(j * K, 8)
            pltpu.sync_copy(rows_v.at[b], acc.at[dst_v.at[pl.ds(jb, K)]],
                            add=True)

        e0 = wid * EPW
        for p in range(NPH):
            off = pl.multiple_of(e0 + p * NCHP * K, 8)
            pltpu.sync_copy(src_hbm.at[pl.ds(off, NCHP * K)], src_v)
            pltpu.sync_copy(dst_hbm.at[pl.ds(off, NCHP * K)], dst_v)
            for b in range(NBUF):
                fire_gather(b, b)
            if p == 0:
                pltpu.make_async_copy(P_hbm.at[pl.ds(r0, RPT)],
                                      acc.at[pl.ds(r0, RPT)], isem).wait()

                @pl.when(sid == 15)
                def _():
                    pltpu.make_async_copy(
                        P_hbm.at[pl.ds(16 * RPT, N - 16 * RPT)],
                        acc.at[pl.ds(16 * RPT, N - 16 * RPT)], isem).wait()

                plsc.subcore_barrier()

            def body(jo, carry):
                for b in range(NBUF):
                    j = jo * NBUF + b
                    wait_gather(j, b)
                    scatter(j, b)

                    @pl.when(j + NBUF < NCHP)
                    def _():
                        fire_gather(j + NBUF, b)
                return carry

            lax.fori_loop(0, NCHP // NBUF, body, 0)
        plsc.subcore_barrier()
        pltpu.sync_copy(acc.at[pl.ds(r0, RPT)], m_hbm.at[cid, pl.ds(r0, RPT)])

        @pl.when(sid == 15)
        def _():
            pltpu.sync_copy(acc.at[pl.ds(16 * RPT, N - 16 * RPT)],
                            m_hbm.at[cid, pl.ds(16 * RPT, N - 16 * RPT)])

    return scatter_kernel


@functools.lru_cache(maxsize=None)
def _get_scatter():
    return _make_scatter()


def _scatter_edges(P, src_p, dst_p):
    return _get_scatter()(P, src_p, dst_p)


def kernel(x, edge_index, batch, enc_W, enc_b, msg_W1, msg_b1, msg_W2, msg_b2,
           gru_Wih, gru_bih, gru_Whh, gru_bhh, head_W1, head_b1, head_W2,
           head_b2):
    pad = E_PAD - E
    pidx = jnp.arange(pad, dtype=jnp.int32)
    src_p = jnp.concatenate([edge_index[0], N + (pidx % (N_P - N))])
    dst_p = jnp.concatenate([edge_index[1], pidx % N])

    h, P = _enc_msg(x, enc_W, enc_b.reshape(1, H),
                    msg_W1[0], msg_b1[0].reshape(1, H),
                    msg_W2[0], msg_b2[0].reshape(1, H))
    for l in range(L):
        m = _scatter_edges(P, src_p, dst_p)
        gru_args = (m[0], m[1], P, h,
                    gru_Wih[l], gru_bih[l].reshape(1, 3 * H),
                    gru_Whh[l], gru_bhh[l].reshape(1, 3 * H))
        if l < L - 1:
            h, P = _gru_msg(*gru_args,
                            msg_W1[l + 1], msg_b1[l + 1].reshape(1, H),
                            msg_W2[l + 1], msg_b2[l + 1].reshape(1, H))
        else:
            out = _gru_pool(*gru_args, batch.reshape(N, 1),
                            head_W1, head_b1.reshape(1, H // 2),
                            head_W2, head_b2.reshape(1, 1))
    return out.reshape(G)
